# baseline (device time: 63331 ns/iter reference)
import jax
import jax.numpy as jnp
from jax import lax
from jax.experimental import pallas as pl
from jax.experimental.pallas import tpu as pltpu


def kernel(Q, K, V, bt, lens):
    B, _, H, D = Q.shape
    P, BS, _, _ = K.shape
    NK = P * BS
    S_slots = bt.shape[1]
    scale = D ** -0.5

    Q2 = Q.reshape(B, H * D)
    K2 = K.reshape(NK, H * D)
    V2 = V.reshape(NK, H * D)
    lens2 = lens.reshape(B, 1)

    CW = 2 * D

    def body(q_ref, k_ref, v_ref, bt_ref, lens_ref, out_ref,
             comm_send, comm_recv, send_sem, recv_sem):
        my_x = lax.axis_index("x")
        my_y = lax.axis_index("y")

        barrier_sem = pltpu.get_barrier_semaphore()
        pl.semaphore_signal(
            barrier_sem, inc=1,
            device_id=(1 - my_x, my_y),
            device_id_type=pl.DeviceIdType.MESH,
        )
        pl.semaphore_wait(barrier_sem, 1)

        bt_v = bt_ref[:, :]
        lens_v = lens_ref[:, :]
        slot_i = lax.broadcasted_iota(jnp.int32, (B, S_slots, P), 1)
        page_i = lax.broadcasted_iota(jnp.int32, (B, S_slots, P), 2)
        valid = slot_i < lens_v[:, :, None]
        eq = bt_v[:, :, None] == (page_i + my_x * P)
        counts = jnp.sum(
            jnp.where(valid & eq, 1.0, 0.0), axis=1
        )

        k_page = lax.broadcasted_iota(jnp.int32, (P, NK), 1) // BS
        p_row = lax.broadcasted_iota(jnp.int32, (P, NK), 0)
        E = jnp.where(k_page == p_row, 1.0, 0.0)
        counts_k = lax.dot_general(
            counts, E, (((1,), (0,)), ((), ())),
            preferred_element_type=jnp.float32,
        )
        key_mask = counts_k > 0.0

        for h in range(H):
            qh = q_ref[:, h * D:(h + 1) * D].astype(jnp.bfloat16)
            kh = k_ref[:, h * D:(h + 1) * D].astype(jnp.bfloat16)
            vh = v_ref[:, h * D:(h + 1) * D].astype(jnp.bfloat16)
            s_mat = lax.dot_general(
                qh, kh, (((1,), (1,)), ((), ())),
                preferred_element_type=jnp.float32,
            ) * scale
            s_mat = jnp.where(key_mask, s_mat, -1e30)
            m = jnp.max(s_mat, axis=1, keepdims=True)
            p = counts_k * jnp.exp(s_mat - m)
            ssum = jnp.sum(p, axis=1, keepdims=True)
            o = lax.dot_general(
                p.astype(jnp.bfloat16), vh, (((1,), (0,)), ((), ())),
                preferred_element_type=jnp.float32,
            )
            comm_send[h, :, 0:D] = o
            comm_send[h, :, D:D + 1] = m
            comm_send[h, :, D + 1:D + 2] = ssum

        rdma = pltpu.make_async_remote_copy(
            src_ref=comm_send,
            dst_ref=comm_recv,
            send_sem=send_sem,
            recv_sem=recv_sem,
            device_id=(1 - my_x, my_y),
            device_id_type=pl.DeviceIdType.MESH,
        )
        rdma.start()
        rdma.wait()

        for h in range(H):
            c1 = comm_send[h]
            c2 = comm_recv[h]
            o1, m1, s1 = c1[:, 0:D], c1[:, D:D + 1], c1[:, D + 1:D + 2]
            o2, m2, s2 = c2[:, 0:D], c2[:, D:D + 1], c2[:, D + 1:D + 2]
            mm = jnp.maximum(m1, m2)
            a1 = jnp.exp(m1 - mm)
            a2 = jnp.exp(m2 - mm)
            res = (a1 * o1 + a2 * o2) / (a1 * s1 + a2 * s2)
            out_ref[:, 0, h, :] = res

    return pl.pallas_call(
        body,
        out_shape=jax.ShapeDtypeStruct((B, 1, H, D), jnp.float32),
        in_specs=[pl.BlockSpec(memory_space=pltpu.VMEM)] * 5,
        out_specs=pl.BlockSpec(memory_space=pltpu.VMEM),
        scratch_shapes=[
            pltpu.VMEM((H, B, CW), jnp.float32),
            pltpu.VMEM((H, B, CW), jnp.float32),
            pltpu.SemaphoreType.DMA,
            pltpu.SemaphoreType.DMA,
        ],
        compiler_params=pltpu.CompilerParams(collective_id=0),
    )(Q2, K2, V2, bt, lens2)


# device time: 57588 ns/iter; 1.0997x vs baseline; 1.0997x over previous
import jax
import jax.numpy as jnp
from jax import lax
from jax.experimental import pallas as pl
from jax.experimental.pallas import tpu as pltpu


def kernel(Q, K, V, bt, lens):
    B, _, H, D = Q.shape
    P, BS, _, _ = K.shape
    NK = P * BS
    S_slots = bt.shape[1]
    scale = D ** -0.5

    lens2 = lens.reshape(B, 1)

    CW = 2 * D

    def body(q_ref, k_hbm, v_hbm, bt_ref, lens_ref, out_ref,
             kbuf, vbuf, comm_send, comm_recv,
             ksem, vsem, send_sem, recv_sem):
        my_x = lax.axis_index("x")
        my_y = lax.axis_index("y")

        def k_copy(h, slot):
            return pltpu.make_async_copy(
                k_hbm.at[:, :, h, :], kbuf.at[slot], ksem.at[slot]
            )

        def v_copy(h, slot):
            return pltpu.make_async_copy(
                v_hbm.at[:, :, h, :], vbuf.at[slot], vsem.at[slot]
            )

        k_copy(0, 0).start()
        v_copy(0, 0).start()

        barrier_sem = pltpu.get_barrier_semaphore()
        pl.semaphore_signal(
            barrier_sem, inc=1,
            device_id=(1 - my_x, my_y),
            device_id_type=pl.DeviceIdType.MESH,
        )
        pl.semaphore_wait(barrier_sem, 1)

        bt_v = bt_ref[:, :]
        lens_v = lens_ref[:, :]
        slot_i = lax.broadcasted_iota(jnp.int32, (B, S_slots, P), 1)
        page_i = lax.broadcasted_iota(jnp.int32, (B, S_slots, P), 2)
        valid = slot_i < lens_v[:, :, None]
        eq = bt_v[:, :, None] == (page_i + my_x * P)
        counts = jnp.sum(
            jnp.where(valid & eq, 1.0, 0.0), axis=1
        )

        k_page = lax.broadcasted_iota(jnp.int32, (P, NK), 1) // BS
        p_row = lax.broadcasted_iota(jnp.int32, (P, NK), 0)
        E = jnp.where(k_page == p_row, 1.0, 0.0)
        counts_k = lax.dot_general(
            counts, E, (((1,), (0,)), ((), ())),
            preferred_element_type=jnp.float32,
        )
        key_mask = counts_k > 0.0

        for h in range(H):
            slot = h % 2
            if h + 1 < H:
                k_copy(h + 1, 1 - slot).start()
                v_copy(h + 1, 1 - slot).start()
            k_copy(h, slot).wait()
            v_copy(h, slot).wait()

            qh = q_ref[:, 0, h, :]
            kh = kbuf[slot].reshape(NK, D)
            vh = vbuf[slot].reshape(NK, D)
            s_mat = lax.dot_general(
                qh, kh, (((1,), (1,)), ((), ())),
                preferred_element_type=jnp.float32,
            ) * scale
            s_mat = jnp.where(key_mask, s_mat, -1e30)
            m = jnp.max(s_mat, axis=1, keepdims=True)
            p = counts_k * jnp.exp(s_mat - m)
            ssum = jnp.sum(p, axis=1, keepdims=True)
            o = lax.dot_general(
                p, vh, (((1,), (0,)), ((), ())),
                preferred_element_type=jnp.float32,
            )
            comm_send[h, :, 0:D] = o
            comm_send[h, :, D:D + 1] = m
            comm_send[h, :, D + 1:D + 2] = ssum

        rdma = pltpu.make_async_remote_copy(
            src_ref=comm_send,
            dst_ref=comm_recv,
            send_sem=send_sem,
            recv_sem=recv_sem,
            device_id=(1 - my_x, my_y),
            device_id_type=pl.DeviceIdType.MESH,
        )
        rdma.start()
        rdma.wait()

        for h in range(H):
            c1 = comm_send[h]
            c2 = comm_recv[h]
            o1, m1, s1 = c1[:, 0:D], c1[:, D:D + 1], c1[:, D + 1:D + 2]
            o2, m2, s2 = c2[:, 0:D], c2[:, D:D + 1], c2[:, D + 1:D + 2]
            mm = jnp.maximum(m1, m2)
            a1 = jnp.exp(m1 - mm)
            a2 = jnp.exp(m2 - mm)
            res = (a1 * o1 + a2 * o2) / (a1 * s1 + a2 * s2)
            out_ref[:, 0, h, :] = res

    return pl.pallas_call(
        body,
        out_shape=jax.ShapeDtypeStruct((B, 1, H, D), jnp.float32),
        in_specs=[
            pl.BlockSpec(memory_space=pltpu.VMEM),
            pl.BlockSpec(memory_space=pl.ANY),
            pl.BlockSpec(memory_space=pl.ANY),
            pl.BlockSpec(memory_space=pltpu.VMEM),
            pl.BlockSpec(memory_space=pltpu.VMEM),
        ],
        out_specs=pl.BlockSpec(memory_space=pltpu.VMEM),
        scratch_shapes=[
            pltpu.VMEM((2, P, BS, D), jnp.float32),
            pltpu.VMEM((2, P, BS, D), jnp.float32),
            pltpu.VMEM((H, B, CW), jnp.float32),
            pltpu.VMEM((H, B, CW), jnp.float32),
            pltpu.SemaphoreType.DMA((2,)),
            pltpu.SemaphoreType.DMA((2,)),
            pltpu.SemaphoreType.DMA,
            pltpu.SemaphoreType.DMA,
        ],
        compiler_params=pltpu.CompilerParams(collective_id=0),
    )(Q, K, V, bt, lens2)


# device time: 21512 ns/iter; 2.9440x vs baseline; 2.6770x over previous
import jax
import jax.numpy as jnp
from jax import lax
from jax.experimental import pallas as pl
from jax.experimental.pallas import tpu as pltpu


def kernel(Q, K, V, bt, lens):
    B, _, H, D = Q.shape
    P, BS, _, _ = K.shape
    NK = P * BS
    S_slots = bt.shape[1]
    scale = D ** -0.5

    Kt = jnp.transpose(K, (1, 2, 3, 0))
    Vt = jnp.transpose(V, (1, 2, 3, 0))
    lens2 = lens.reshape(B, 1)

    CW = 2 * D

    def body(q_ref, kt_ref, vt_ref, bt_ref, lens_ref, out_ref,
             comm_send, comm_recv, send_sem, recv_sem):
        my_x = lax.axis_index("x")
        my_y = lax.axis_index("y")

        barrier_sem = pltpu.get_barrier_semaphore()
        pl.semaphore_signal(
            barrier_sem, inc=1,
            device_id=(1 - my_x, my_y),
            device_id_type=pl.DeviceIdType.MESH,
        )
        pl.semaphore_wait(barrier_sem, 1)

        bt_v = bt_ref[:, :]
        lens_v = lens_ref[:, :]
        slot_i = lax.broadcasted_iota(jnp.int32, (B, S_slots, P), 1)
        page_i = lax.broadcasted_iota(jnp.int32, (B, S_slots, P), 2)
        valid = slot_i < lens_v[:, :, None]
        eq = bt_v[:, :, None] == (page_i + my_x * P)
        counts = jnp.sum(
            jnp.where(valid & eq, 1.0, 0.0), axis=1
        )

        k_page = lax.broadcasted_iota(jnp.int32, (P, NK), 1) % P
        p_row = lax.broadcasted_iota(jnp.int32, (P, NK), 0)
        E = jnp.where(k_page == p_row, 1.0, 0.0)
        counts_k = lax.dot_general(
            counts, E, (((1,), (0,)), ((), ())),
            preferred_element_type=jnp.float32,
        )
        key_mask = counts_k > 0.0

        for h in range(H):
            qh = q_ref[:, 0, h, :]
            kt_h = kt_ref[:, h, :, :]
            G = jnp.concatenate(
                [kt_h[s] for s in range(BS)], axis=1
            )
            s_mat = lax.dot_general(
                qh, G, (((1,), (0,)), ((), ())),
                preferred_element_type=jnp.float32,
            ) * scale
            s_mat = jnp.where(key_mask, s_mat, -1e30)
            m = jnp.max(s_mat, axis=1, keepdims=True)
            p = counts_k * jnp.exp(s_mat - m)
            ssum = jnp.sum(p, axis=1, keepdims=True)
            v2_h = jnp.swapaxes(
                vt_ref[:, h, :, :], 1, 2
            ).reshape(NK, D)
            o = lax.dot_general(
                p, v2_h, (((1,), (0,)), ((), ())),
                preferred_element_type=jnp.float32,
            )
            comm_send[h, :, 0:D] = o
            comm_send[h, :, D:D + 1] = m
            comm_send[h, :, D + 1:D + 2] = ssum

        rdma = pltpu.make_async_remote_copy(
            src_ref=comm_send,
            dst_ref=comm_recv,
            send_sem=send_sem,
            recv_sem=recv_sem,
            device_id=(1 - my_x, my_y),
            device_id_type=pl.DeviceIdType.MESH,
        )
        rdma.start()
        rdma.wait()

        for h in range(H):
            c1 = comm_send[h]
            c2 = comm_recv[h]
            o1, m1, s1 = c1[:, 0:D], c1[:, D:D + 1], c1[:, D + 1:D + 2]
            o2, m2, s2 = c2[:, 0:D], c2[:, D:D + 1], c2[:, D + 1:D + 2]
            mm = jnp.maximum(m1, m2)
            a1 = jnp.exp(m1 - mm)
            a2 = jnp.exp(m2 - mm)
            res = (a1 * o1 + a2 * o2) / (a1 * s1 + a2 * s2)
            out_ref[:, 0, h, :] = res

    return pl.pallas_call(
        body,
        out_shape=jax.ShapeDtypeStruct((B, 1, H, D), jnp.float32),
        in_specs=[pl.BlockSpec(memory_space=pltpu.VMEM)] * 5,
        out_specs=pl.BlockSpec(memory_space=pltpu.VMEM),
        scratch_shapes=[
            pltpu.VMEM((H, B, CW), jnp.float32),
            pltpu.VMEM((H, B, CW), jnp.float32),
            pltpu.SemaphoreType.DMA,
            pltpu.SemaphoreType.DMA,
        ],
        compiler_params=pltpu.CompilerParams(collective_id=0),
    )(Q, Kt, Vt, bt, lens2)
